# baseline (speedup 1.0000x reference)
import functools

import jax
import jax.numpy as jnp
from jax import lax
from jax.experimental import pallas as pl
from jax.experimental.pallas import tpu as pltpu
from jax.experimental.pallas import tpu_sc as plsc

NC = 2
NS = 16
NW = NC * NS

N = 10000
NP = 10240
NPR = NP // 128
E = 320000
EPT = E // NW
CHUNK = 80
NITER = EPT // CHUNK
NPAIRS = NITER // 2
RPT = NP // NS
NB = 128
NGRID = (N + NB - 1) // NB


def _mesh():
    return plsc.VectorSubcoreMesh(
        core_axis_name="c", subcore_axis_name="s", num_cores=NC, num_subcores=NS
    )


def _sc_hist(col):
    @functools.partial(
        pl.kernel,
        out_type=jax.ShapeDtypeStruct((NW, NP), jnp.float32),
        mesh=_mesh(),
        compiler_params=pltpu.CompilerParams(needs_layout_passes=False),
        scratch_types=[
            pltpu.VMEM((EPT,), jnp.int32),
            pltpu.VMEM((NP,), jnp.float32),
        ],
    )
    def k(col_hbm, out_hbm, col_v, hist_v):
        cid = lax.axis_index("c")
        sid = lax.axis_index("s")
        wid = cid * NS + sid

        def zbody(i, carry):
            hist_v[pl.ds(i * 16, 16)] = jnp.zeros((16,), jnp.float32)
            return carry

        lax.fori_loop(0, NP // 16, zbody, 0)
        pltpu.sync_copy(col_hbm.at[pl.ds(wid * EPT, EPT)], col_v)
        ones = jnp.ones((16,), jnp.float32)

        def hbody(i, carry):
            idx = col_v[pl.ds(i * 16, 16)]
            plsc.addupdate_scatter(hist_v, [idx], ones)
            return carry

        lax.fori_loop(0, EPT // 16, hbody, 0)
        pltpu.sync_copy(hist_v, out_hbm.at[wid])

    return k(col)


def _sc_scatter(adj4d, g):
    @functools.partial(
        pl.kernel,
        out_type=jax.ShapeDtypeStruct((NC, NP, 128), jnp.float32),
        mesh=_mesh(),
        compiler_params=pltpu.CompilerParams(
            needs_layout_passes=False, use_tc_tiling_on_sc=False
        ),
        scratch_types=[
            pltpu.VMEM((NITER, CHUNK), jnp.int32),
            pltpu.VMEM((NITER, CHUNK), jnp.int32),
            pltpu.VMEM((CHUNK, 128), jnp.float32),
            pltpu.VMEM((CHUNK, 128), jnp.float32),
            pltpu.VMEM_SHARED((NP, 128), jnp.float32),
            pltpu.SemaphoreType.DMA,
            pltpu.SemaphoreType.DMA,
        ],
    )
    def k(adj_hbm, g_hbm, out_hbm, ridx, cidx, buf0, buf1, acc, s0, s1):
        cid = lax.axis_index("c")
        sid = lax.axis_index("s")
        wid = cid * NS + sid

        def zbody(i, carry):
            r = i // 8
            cc = (i % 8) * 16
            buf0[r, pl.ds(cc, 16)] = jnp.zeros((16,), jnp.float32)
            return carry

        lax.fori_loop(0, CHUNK * 8, zbody, 0)
        for b in range(RPT // CHUNK):
            pltpu.sync_copy(buf0, acc.at[pl.ds(sid * RPT + b * CHUNK, CHUNK), :])
        pltpu.sync_copy(adj_hbm.at[0, wid], ridx)
        pltpu.sync_copy(adj_hbm.at[1, wid], cidx)
        plsc.subcore_barrier()

        pltpu.async_copy(g_hbm.at[ridx.at[0]], buf0, s0)

        def body(j, carry):
            i0 = 2 * j
            i1 = 2 * j + 1
            pltpu.make_async_copy(g_hbm.at[ridx.at[i0]], buf0, s0).wait()
            pltpu.async_copy(g_hbm.at[ridx.at[i1]], buf1, s1)
            pltpu.sync_copy(buf0, acc.at[cidx.at[i0]], add=True)
            pltpu.make_async_copy(g_hbm.at[ridx.at[i1]], buf1, s1).wait()
            pltpu.async_copy(g_hbm.at[ridx.at[i0 + 2]], buf0, s0)
            pltpu.sync_copy(buf1, acc.at[cidx.at[i1]], add=True)
            return carry

        lax.fori_loop(0, NPAIRS, body, 0)
        pltpu.make_async_copy(g_hbm.at[ridx.at[NITER - 1]], buf0, s0).wait()
        pltpu.sync_copy(buf0, acc.at[cidx.at[NITER - 1]], add=True)
        plsc.subcore_barrier()
        pltpu.sync_copy(
            acc.at[pl.ds(sid * RPT, RPT), :],
            out_hbm.at[cid, pl.ds(sid * RPT, RPT), :],
        )

    return k(adj4d, g)


def _colvec(v):
    ii = lax.broadcasted_iota(jnp.int32, (128, 128), 0)
    jj = lax.broadcasted_iota(jnp.int32, (128, 128), 1)
    d = jnp.where(ii == jj, jnp.broadcast_to(v, (128, 128)), 0.0)
    return jnp.sum(d, axis=1, keepdims=True)


def _rowmask(i, v):
    rid = i * NB + lax.broadcasted_iota(jnp.int32, v.shape, 0)
    return jnp.where(rid < N, v, 0.0)


def _tc_g(parts, x, w):
    def body(p_ref, x_ref, w_ref, o_ref):
        i = pl.program_id(0)
        p = p_ref[:, pl.ds(i * NB, NB)]
        d = jnp.sum(p, axis=0, keepdims=True) + 1.0
        dinv = lax.rsqrt(d)
        h = jnp.dot(
            _rowmask(i, x_ref[...]), w_ref[...], preferred_element_type=jnp.float32
        )
        o_ref[...] = h * _colvec(dinv)

    return pl.pallas_call(
        body,
        grid=(NGRID,),
        in_specs=[
            pl.BlockSpec((NW, NP), lambda i: (0, 0)),
            pl.BlockSpec((NB, 128), lambda i: (i, 0)),
            pl.BlockSpec((128, 128), lambda i: (0, 0)),
        ],
        out_specs=pl.BlockSpec((NB, 128), lambda i: (i, 0)),
        out_shape=jax.ShapeDtypeStruct((N, 128), jnp.float32),
    )(parts, x, w)


def _tc_out(sparts, g, degparts, bc, pa, wl, bl):
    def body(s_ref, g_ref, p_ref, bc_ref, pa_ref, wl_ref, bl_ref, o_ref):
        i = pl.program_id(0)
        p = p_ref[:, pl.ds(i * NB, NB)]
        d = jnp.sum(p, axis=0, keepdims=True) + 1.0
        dinv = lax.rsqrt(d)
        s = _rowmask(i, s_ref[0] + s_ref[1] + g_ref[...])
        agg = s * _colvec(dinv) + bc_ref[...]
        a = pa_ref[0, 0]
        act = jnp.where(agg > 0, agg, a * agg)
        o_ref[...] = (
            jnp.dot(act, wl_ref[...], preferred_element_type=jnp.float32) + bl_ref[...]
        )

    return pl.pallas_call(
        body,
        grid=(NGRID,),
        in_specs=[
            pl.BlockSpec((NC, NB, 128), lambda i: (0, i, 0)),
            pl.BlockSpec((NB, 128), lambda i: (i, 0)),
            pl.BlockSpec((NW, NP), lambda i: (0, 0)),
            pl.BlockSpec((1, 128), lambda i: (0, 0)),
            pl.BlockSpec((1, 1), lambda i: (0, 0)),
            pl.BlockSpec((128, 128), lambda i: (0, 0)),
            pl.BlockSpec((1, 128), lambda i: (0, 0)),
        ],
        out_specs=pl.BlockSpec((NB, 128), lambda i: (i, 0)),
        out_shape=jax.ShapeDtypeStruct((N, 128), jnp.float32),
    )(sparts, g, degparts, bc, pa, wl, bl)


def kernel(x, adj, W_conv, b_conv, prelu_a, W_lin, b_lin):
    row = adj[0]
    col = adj[1]
    degparts = _sc_hist(col)
    g = _tc_g(degparts, x, W_conv)
    sparts = _sc_scatter(adj.reshape(2, NW, NITER, CHUNK), g)
    return _tc_out(
        sparts,
        g,
        degparts,
        b_conv.reshape(1, 128),
        jnp.asarray(prelu_a, jnp.float32).reshape(1, 1),
        W_lin,
        b_lin.reshape(1, 128),
    )


# speedup vs baseline: 1.3276x; 1.3276x over previous
import functools

import jax
import jax.numpy as jnp
from jax import lax
from jax.experimental import pallas as pl
from jax.experimental.pallas import tpu as pltpu
from jax.experimental.pallas import tpu_sc as plsc

NC = 2
NS = 16
NW = NC * NS

N = 10000
NP = 10240
NPR = NP // 128
E = 320000
EPT = E // NW
CHUNK = 80
NITER = EPT // CHUNK
NPAIRS = NITER // 2
NRD = 10112
RPT = NRD // NS
ZR = RPT // 8
NB = 512
NGRID = (N + NB - 1) // NB


def _mesh():
    return plsc.VectorSubcoreMesh(
        core_axis_name="c", subcore_axis_name="s", num_cores=NC, num_subcores=NS
    )


def _sc_hist(adj4d):
    @functools.partial(
        pl.kernel,
        out_type=jax.ShapeDtypeStruct((NW, NP), jnp.float32),
        mesh=_mesh(),
        compiler_params=pltpu.CompilerParams(
            needs_layout_passes=False, use_tc_tiling_on_sc=False
        ),
        scratch_types=[
            pltpu.VMEM((NITER, CHUNK), jnp.int32),
            pltpu.VMEM((NP,), jnp.float32),
        ],
    )
    def k(adj_hbm, out_hbm, col_v, hist_v):
        cid = lax.axis_index("c")
        sid = lax.axis_index("s")
        wid = cid * NS + sid

        def zbody(i, carry):
            hist_v[pl.ds(i * 16, 16)] = jnp.zeros((16,), jnp.float32)
            return carry

        lax.fori_loop(0, NP // 16, zbody, 0)
        pltpu.sync_copy(adj_hbm.at[1, wid], col_v)
        ones = jnp.ones((16,), jnp.float32)
        LPR = CHUNK // 16

        def hbody(i, carry):
            idx = col_v[i // LPR, pl.ds((i % LPR) * 16, 16)]
            plsc.addupdate_scatter(hist_v, [idx], ones)
            return carry

        lax.fori_loop(0, (EPT // 16), hbody, 0)
        pltpu.sync_copy(hist_v, out_hbm.at[wid])

    return k(adj4d)


def _sc_scatter(adj4d, g):
    @functools.partial(
        pl.kernel,
        out_type=jax.ShapeDtypeStruct((NC, NRD, 128), jnp.float32),
        mesh=_mesh(),
        compiler_params=pltpu.CompilerParams(
            needs_layout_passes=False, use_tc_tiling_on_sc=False
        ),
        scratch_types=[
            pltpu.VMEM((NITER, CHUNK), jnp.int32),
            pltpu.VMEM((NITER, CHUNK), jnp.int32),
            pltpu.VMEM((CHUNK, 128), jnp.float32),
            pltpu.VMEM((CHUNK, 128), jnp.float32),
            pltpu.VMEM_SHARED((NRD, 128), jnp.float32),
            pltpu.SemaphoreType.DMA,
            pltpu.SemaphoreType.DMA,
            pltpu.SemaphoreType.DMA,
            pltpu.SemaphoreType.DMA,
        ],
    )
    def k(adj_hbm, g_hbm, out_hbm, ridx, cidx, buf0, buf1, acc, s0, s1, t0, t1):
        cid = lax.axis_index("c")
        sid = lax.axis_index("s")
        wid = cid * NS + sid

        def zbody(i, carry):
            r = i // 8
            cc = (i % 8) * 16
            buf0[r, pl.ds(cc, 16)] = jnp.zeros((16,), jnp.float32)
            return carry

        lax.fori_loop(0, CHUNK * 8, zbody, 0)
        for b in range(8):
            pltpu.sync_copy(
                buf0.at[pl.ds(0, ZR), :], acc.at[pl.ds(sid * RPT + b * ZR, ZR), :]
            )
        pltpu.sync_copy(adj_hbm.at[0, wid], ridx)
        pltpu.sync_copy(adj_hbm.at[1, wid], cidx)
        plsc.subcore_barrier()

        pltpu.async_copy(g_hbm.at[ridx.at[0]], buf0, s0)
        pltpu.async_copy(g_hbm.at[ridx.at[1]], buf1, s1)

        def body(j, carry):
            i0 = 2 * j
            i1 = 2 * j + 1
            pltpu.make_async_copy(g_hbm.at[ridx.at[i0]], buf0, s0).wait()
            pltpu.async_copy(buf0, acc.at[cidx.at[i0]], t0, add=True)
            pltpu.make_async_copy(g_hbm.at[ridx.at[i1]], buf1, s1).wait()
            pltpu.async_copy(buf1, acc.at[cidx.at[i1]], t1, add=True)
            pltpu.make_async_copy(buf0, acc.at[cidx.at[i0]], t0).wait()
            pltpu.async_copy(g_hbm.at[ridx.at[i0 + 2]], buf0, s0)
            pltpu.make_async_copy(buf1, acc.at[cidx.at[i1]], t1).wait()

            @pl.when(i1 + 2 < NITER)
            def _():
                pltpu.async_copy(g_hbm.at[ridx.at[i1 + 2]], buf1, s1)

            return carry

        lax.fori_loop(0, NPAIRS, body, 0)
        pltpu.make_async_copy(g_hbm.at[ridx.at[NITER - 1]], buf0, s0).wait()
        pltpu.sync_copy(buf0, acc.at[cidx.at[NITER - 1]], add=True)
        plsc.subcore_barrier()
        pltpu.sync_copy(
            acc.at[pl.ds(sid * RPT, RPT), :],
            out_hbm.at[cid, pl.ds(sid * RPT, RPT), :],
        )

    return k(adj4d, g)


def _colvec(v):
    ii = lax.broadcasted_iota(jnp.int32, (128, 128), 0)
    jj = lax.broadcasted_iota(jnp.int32, (128, 128), 1)
    d = jnp.where(ii == jj, jnp.broadcast_to(v, (128, 128)), 0.0)
    return jnp.sum(d, axis=1, keepdims=True)


def _colvec_block(v):
    return jnp.concatenate(
        [_colvec(v[:, k * 128 : (k + 1) * 128]) for k in range(NB // 128)], axis=0
    )


def _rowmask(i, v):
    rid = i * NB + lax.broadcasted_iota(jnp.int32, v.shape, 0)
    return jnp.where(rid < N, v, 0.0)


def _tc_g(parts, x, w):
    def body(p_ref, x_ref, w_ref, o_ref):
        i = pl.program_id(0)
        p = p_ref[:, pl.ds(i * NB, NB)]
        d = jnp.sum(p, axis=0, keepdims=True) + 1.0
        dinv = lax.rsqrt(d)
        h = jnp.dot(
            _rowmask(i, x_ref[...]), w_ref[...], preferred_element_type=jnp.float32
        )
        o_ref[...] = h * _colvec_block(dinv)

    return pl.pallas_call(
        body,
        grid=(NGRID,),
        in_specs=[
            pl.BlockSpec((NW, NP), lambda i: (0, 0)),
            pl.BlockSpec((NB, 128), lambda i: (i, 0)),
            pl.BlockSpec((128, 128), lambda i: (0, 0)),
        ],
        out_specs=pl.BlockSpec((NB, 128), lambda i: (i, 0)),
        out_shape=jax.ShapeDtypeStruct((N, 128), jnp.float32),
    )(parts, x, w)


def _tc_out(sparts, g, degparts, bc, pa, wl, bl):
    def body(s_ref, g_ref, p_ref, bc_ref, pa_ref, wl_ref, bl_ref, o_ref):
        i = pl.program_id(0)
        p = p_ref[:, pl.ds(i * NB, NB)]
        d = jnp.sum(p, axis=0, keepdims=True) + 1.0
        dinv = lax.rsqrt(d)
        s = _rowmask(i, s_ref[0] + s_ref[1] + g_ref[...])
        agg = s * _colvec_block(dinv) + bc_ref[...]
        a = pa_ref[0, 0]
        act = jnp.where(agg > 0, agg, a * agg)
        o_ref[...] = (
            jnp.dot(act, wl_ref[...], preferred_element_type=jnp.float32) + bl_ref[...]
        )

    return pl.pallas_call(
        body,
        grid=(NGRID,),
        in_specs=[
            pl.BlockSpec((NC, NB, 128), lambda i: (0, i, 0)),
            pl.BlockSpec((NB, 128), lambda i: (i, 0)),
            pl.BlockSpec((NW, NP), lambda i: (0, 0)),
            pl.BlockSpec((1, 128), lambda i: (0, 0)),
            pl.BlockSpec((1, 1), lambda i: (0, 0)),
            pl.BlockSpec((128, 128), lambda i: (0, 0)),
            pl.BlockSpec((1, 128), lambda i: (0, 0)),
        ],
        out_specs=pl.BlockSpec((NB, 128), lambda i: (i, 0)),
        out_shape=jax.ShapeDtypeStruct((N, 128), jnp.float32),
    )(sparts, g, degparts, bc, pa, wl, bl)


def kernel(x, adj, W_conv, b_conv, prelu_a, W_lin, b_lin):
    row = adj[0]
    col = adj[1]
    adj4d = adj.reshape(2, NW, NITER, CHUNK)
    degparts = _sc_hist(adj4d)
    g = _tc_g(degparts, x, W_conv)
    sparts = _sc_scatter(adj4d, g)
    return _tc_out(
        sparts,
        g,
        degparts,
        b_conv.reshape(1, 128),
        jnp.asarray(prelu_a, jnp.float32).reshape(1, 1),
        W_lin,
        b_lin.reshape(1, 128),
    )


# speedup vs baseline: 1.3437x; 1.0121x over previous
import functools

import jax
import jax.numpy as jnp
from jax import lax
from jax.experimental import pallas as pl
from jax.experimental.pallas import tpu as pltpu
from jax.experimental.pallas import tpu_sc as plsc

NC = 2
NS = 16
NW = NC * NS

N = 10000
NP = 10240
NPR = NP // 128
E = 320000
EPT = E // NW
CHUNK = 100
NITER = EPT // CHUNK
NPAIRS = NITER // 2
NRD = 10112
RPT = NRD // NS
ZR = RPT // 8
NB = 512
NGRID = (N + NB - 1) // NB


def _mesh():
    return plsc.VectorSubcoreMesh(
        core_axis_name="c", subcore_axis_name="s", num_cores=NC, num_subcores=NS
    )


def _sc_hist(adj):
    @functools.partial(
        pl.kernel,
        out_type=jax.ShapeDtypeStruct((NW, NP), jnp.float32),
        mesh=_mesh(),
        compiler_params=pltpu.CompilerParams(
            needs_layout_passes=False, use_tc_tiling_on_sc=False
        ),
        scratch_types=[
            pltpu.VMEM((EPT,), jnp.int32),
            pltpu.VMEM((NP,), jnp.float32),
        ],
    )
    def k(adj_hbm, out_hbm, col_v, hist_v):
        cid = lax.axis_index("c")
        sid = lax.axis_index("s")
        wid = cid * NS + sid

        def zbody(i, carry):
            hist_v[pl.ds(i * 16, 16)] = jnp.zeros((16,), jnp.float32)
            return carry

        lax.fori_loop(0, NP // 16, zbody, 0)
        pltpu.sync_copy(adj_hbm.at[1, pl.ds(wid * EPT, EPT)], col_v)
        ones = jnp.ones((16,), jnp.float32)

        def hbody(i, carry):
            idx = col_v[pl.ds(i * 16, 16)]
            plsc.addupdate_scatter(hist_v, [idx], ones)
            return carry

        lax.fori_loop(0, (EPT // 16), hbody, 0)
        pltpu.sync_copy(hist_v, out_hbm.at[wid])

    return k(adj)


def _sc_scatter(adj4d, g):
    @functools.partial(
        pl.kernel,
        out_type=jax.ShapeDtypeStruct((NC, NRD, 128), jnp.float32),
        mesh=_mesh(),
        compiler_params=pltpu.CompilerParams(
            needs_layout_passes=False, use_tc_tiling_on_sc=False
        ),
        scratch_types=[
            pltpu.VMEM((NITER, CHUNK), jnp.int32),
            pltpu.VMEM((NITER, CHUNK), jnp.int32),
            pltpu.VMEM((CHUNK, 128), jnp.float32),
            pltpu.VMEM((CHUNK, 128), jnp.float32),
            pltpu.VMEM_SHARED((NRD, 128), jnp.float32),
            pltpu.SemaphoreType.DMA,
            pltpu.SemaphoreType.DMA,
            pltpu.SemaphoreType.DMA,
            pltpu.SemaphoreType.DMA,
        ],
    )
    def k(adj_hbm, g_hbm, out_hbm, ridx, cidx, buf0, buf1, acc, s0, s1, t0, t1):
        cid = lax.axis_index("c")
        sid = lax.axis_index("s")
        wid = cid * NS + sid

        def zbody(i, carry):
            r = i // 8
            cc = (i % 8) * 16
            buf0[r, pl.ds(cc, 16)] = jnp.zeros((16,), jnp.float32)
            return carry

        lax.fori_loop(0, CHUNK * 8, zbody, 0)
        for b in range(8):
            pltpu.sync_copy(
                buf0.at[pl.ds(0, ZR), :], acc.at[pl.ds(sid * RPT + b * ZR, ZR), :]
            )
        pltpu.sync_copy(adj_hbm.at[0, wid], ridx)
        pltpu.sync_copy(adj_hbm.at[1, wid], cidx)
        plsc.subcore_barrier()

        pltpu.async_copy(g_hbm.at[ridx.at[0]], buf0, s0)
        pltpu.async_copy(g_hbm.at[ridx.at[1]], buf1, s1)

        def body(j, carry):
            i0 = 2 * j
            i1 = 2 * j + 1
            pltpu.make_async_copy(g_hbm.at[ridx.at[i0]], buf0, s0).wait()
            pltpu.async_copy(buf0, acc.at[cidx.at[i0]], t0, add=True)
            pltpu.make_async_copy(g_hbm.at[ridx.at[i1]], buf1, s1).wait()
            pltpu.async_copy(buf1, acc.at[cidx.at[i1]], t1, add=True)
            pltpu.make_async_copy(buf0, acc.at[cidx.at[i0]], t0).wait()

            @pl.when(i0 + 2 < NITER)
            def _():
                pltpu.async_copy(g_hbm.at[ridx.at[i0 + 2]], buf0, s0)

            pltpu.make_async_copy(buf1, acc.at[cidx.at[i1]], t1).wait()

            @pl.when(i1 + 2 < NITER)
            def _():
                pltpu.async_copy(g_hbm.at[ridx.at[i1 + 2]], buf1, s1)

            return carry

        lax.fori_loop(0, NPAIRS, body, 0)
        plsc.subcore_barrier()
        pltpu.sync_copy(
            acc.at[pl.ds(sid * RPT, RPT), :],
            out_hbm.at[cid, pl.ds(sid * RPT, RPT), :],
        )

    return k(adj4d, g)


def _colvec(v):
    ii = lax.broadcasted_iota(jnp.int32, (128, 128), 0)
    jj = lax.broadcasted_iota(jnp.int32, (128, 128), 1)
    d = jnp.where(ii == jj, jnp.broadcast_to(v, (128, 128)), 0.0)
    return jnp.sum(d, axis=1, keepdims=True)


def _colvec_block(v):
    return jnp.concatenate(
        [_colvec(v[:, k * 128 : (k + 1) * 128]) for k in range(NB // 128)], axis=0
    )


def _rowmask(i, v):
    rid = i * NB + lax.broadcasted_iota(jnp.int32, v.shape, 0)
    return jnp.where(rid < N, v, 0.0)


def _tc_g(parts, x, w):
    def body(p_ref, x_ref, w_ref, o_ref):
        i = pl.program_id(0)
        p = p_ref[:, pl.ds(i * NB, NB)]
        d = jnp.sum(p, axis=0, keepdims=True) + 1.0
        dinv = lax.rsqrt(d)
        h = jnp.dot(
            _rowmask(i, x_ref[...]), w_ref[...], preferred_element_type=jnp.float32
        )
        o_ref[...] = h * _colvec_block(dinv)

    return pl.pallas_call(
        body,
        grid=(NGRID,),
        in_specs=[
            pl.BlockSpec((NW, NP), lambda i: (0, 0)),
            pl.BlockSpec((NB, 128), lambda i: (i, 0)),
            pl.BlockSpec((128, 128), lambda i: (0, 0)),
        ],
        out_specs=pl.BlockSpec((NB, 128), lambda i: (i, 0)),
        out_shape=jax.ShapeDtypeStruct((N, 128), jnp.float32),
    )(parts, x, w)


def _tc_out(sparts, g, degparts, bc, pa, wl, bl):
    def body(s_ref, g_ref, p_ref, bc_ref, pa_ref, wl_ref, bl_ref, o_ref):
        i = pl.program_id(0)
        p = p_ref[:, pl.ds(i * NB, NB)]
        d = jnp.sum(p, axis=0, keepdims=True) + 1.0
        dinv = lax.rsqrt(d)
        s = _rowmask(i, s_ref[0] + s_ref[1] + g_ref[...])
        agg = s * _colvec_block(dinv) + bc_ref[...]
        a = pa_ref[0, 0]
        act = jnp.where(agg > 0, agg, a * agg)
        o_ref[...] = (
            jnp.dot(act, wl_ref[...], preferred_element_type=jnp.float32) + bl_ref[...]
        )

    return pl.pallas_call(
        body,
        grid=(NGRID,),
        in_specs=[
            pl.BlockSpec((NC, NB, 128), lambda i: (0, i, 0)),
            pl.BlockSpec((NB, 128), lambda i: (i, 0)),
            pl.BlockSpec((NW, NP), lambda i: (0, 0)),
            pl.BlockSpec((1, 128), lambda i: (0, 0)),
            pl.BlockSpec((1, 1), lambda i: (0, 0)),
            pl.BlockSpec((128, 128), lambda i: (0, 0)),
            pl.BlockSpec((1, 128), lambda i: (0, 0)),
        ],
        out_specs=pl.BlockSpec((NB, 128), lambda i: (i, 0)),
        out_shape=jax.ShapeDtypeStruct((N, 128), jnp.float32),
    )(sparts, g, degparts, bc, pa, wl, bl)


def kernel(x, adj, W_conv, b_conv, prelu_a, W_lin, b_lin):
    row = adj[0]
    col = adj[1]
    degparts = _sc_hist(adj)
    g = _tc_g(degparts, x, W_conv)
    sparts = _sc_scatter(adj.reshape(2, NW, NITER, CHUNK), g)
    return _tc_out(
        sparts,
        g,
        degparts,
        b_conv.reshape(1, 128),
        jnp.asarray(prelu_a, jnp.float32).reshape(1, 1),
        W_lin,
        b_lin.reshape(1, 128),
    )


# speedup vs baseline: 1.5256x; 1.1353x over previous
import functools

import jax
import jax.numpy as jnp
from jax import lax
from jax.experimental import pallas as pl
from jax.experimental.pallas import tpu as pltpu
from jax.experimental.pallas import tpu_sc as plsc

NC = 2
NS = 16
NW = NC * NS

N = 10000
NP = 10240
NPR = NP // 128
E = 320000
EPT = E // NW
CHUNK = 100
NITER = EPT // CHUNK
NPAIRS = NITER // 2
NRD = 10112
RPT = NRD // NS
ZR = RPT // 8
NB = 512
NGRID = (N + NB - 1) // NB


def _mesh():
    return plsc.VectorSubcoreMesh(
        core_axis_name="c", subcore_axis_name="s", num_cores=NC, num_subcores=NS
    )


def _sc_hist(adj):
    @functools.partial(
        pl.kernel,
        out_type=jax.ShapeDtypeStruct((NW, NP), jnp.float32),
        mesh=_mesh(),
        compiler_params=pltpu.CompilerParams(
            needs_layout_passes=False, use_tc_tiling_on_sc=False
        ),
        scratch_types=[
            pltpu.VMEM((EPT,), jnp.int32),
            pltpu.VMEM((NP,), jnp.float32),
        ],
    )
    def k(adj_hbm, out_hbm, col_v, hist_v):
        cid = lax.axis_index("c")
        sid = lax.axis_index("s")
        wid = cid * NS + sid

        def zbody(i, carry):
            hist_v[pl.ds(i * 16, 16)] = jnp.zeros((16,), jnp.float32)
            return carry

        lax.fori_loop(0, NP // 16, zbody, 0)
        pltpu.sync_copy(adj_hbm.at[1, pl.ds(wid * EPT, EPT)], col_v)
        ones = jnp.ones((16,), jnp.float32)

        def hbody(i, carry):
            idx = col_v[pl.ds(i * 16, 16)]
            plsc.addupdate_scatter(hist_v, [idx], ones)
            return carry

        lax.fori_loop(0, (EPT // 16), hbody, 0)
        pltpu.sync_copy(hist_v, out_hbm.at[wid])

    return k(adj)


def _sc_scatter(adj4d, g):
    @functools.partial(
        pl.kernel,
        out_type=jax.ShapeDtypeStruct((NC, NRD, 128), jnp.bfloat16),
        mesh=_mesh(),
        compiler_params=pltpu.CompilerParams(
            needs_layout_passes=False, use_tc_tiling_on_sc=False
        ),
        scratch_types=[
            pltpu.VMEM((NITER, CHUNK), jnp.int32),
            pltpu.VMEM((NITER, CHUNK), jnp.int32),
            pltpu.VMEM((CHUNK, 128), jnp.bfloat16),
            pltpu.VMEM((CHUNK, 128), jnp.bfloat16),
            pltpu.VMEM_SHARED((NRD, 128), jnp.bfloat16),
            pltpu.SemaphoreType.DMA,
            pltpu.SemaphoreType.DMA,
            pltpu.SemaphoreType.DMA,
            pltpu.SemaphoreType.DMA,
        ],
    )
    def k(adj_hbm, g_hbm, out_hbm, ridx, cidx, buf0, buf1, acc, s0, s1, t0, t1):
        cid = lax.axis_index("c")
        sid = lax.axis_index("s")
        wid = cid * NS + sid

        def zbody(i, carry):
            r = i // 4
            cc = (i % 4) * 32
            buf0[r, pl.ds(cc, 32)] = jnp.zeros((32,), jnp.bfloat16)
            return carry

        lax.fori_loop(0, CHUNK * 4, zbody, 0)
        for b in range(8):
            pltpu.sync_copy(
                buf0.at[pl.ds(0, ZR), :], acc.at[pl.ds(sid * RPT + b * ZR, ZR), :]
            )
        pltpu.sync_copy(adj_hbm.at[0, wid], ridx)
        pltpu.sync_copy(adj_hbm.at[1, wid], cidx)
        plsc.subcore_barrier()

        pltpu.async_copy(g_hbm.at[ridx.at[0]], buf0, s0)
        pltpu.async_copy(g_hbm.at[ridx.at[1]], buf1, s1)

        def body(j, carry):
            i0 = 2 * j
            i1 = 2 * j + 1
            pltpu.make_async_copy(g_hbm.at[ridx.at[i0]], buf0, s0).wait()
            pltpu.async_copy(buf0, acc.at[cidx.at[i0]], t0, add=True)
            pltpu.make_async_copy(g_hbm.at[ridx.at[i1]], buf1, s1).wait()
            pltpu.async_copy(buf1, acc.at[cidx.at[i1]], t1, add=True)
            pltpu.make_async_copy(buf0, acc.at[cidx.at[i0]], t0).wait()

            @pl.when(i0 + 2 < NITER)
            def _():
                pltpu.async_copy(g_hbm.at[ridx.at[i0 + 2]], buf0, s0)

            pltpu.make_async_copy(buf1, acc.at[cidx.at[i1]], t1).wait()

            @pl.when(i1 + 2 < NITER)
            def _():
                pltpu.async_copy(g_hbm.at[ridx.at[i1 + 2]], buf1, s1)

            return carry

        lax.fori_loop(0, NPAIRS, body, 0)
        plsc.subcore_barrier()
        pltpu.sync_copy(
            acc.at[pl.ds(sid * RPT, RPT), :],
            out_hbm.at[cid, pl.ds(sid * RPT, RPT), :],
        )

    return k(adj4d, g)


def _colvec(v):
    ii = lax.broadcasted_iota(jnp.int32, (128, 128), 0)
    jj = lax.broadcasted_iota(jnp.int32, (128, 128), 1)
    d = jnp.where(ii == jj, jnp.broadcast_to(v, (128, 128)), 0.0)
    return jnp.sum(d, axis=1, keepdims=True)


def _colvec_block(v):
    return jnp.concatenate(
        [_colvec(v[:, k * 128 : (k + 1) * 128]) for k in range(NB // 128)], axis=0
    )


def _rowmask(i, v):
    rid = i * NB + lax.broadcasted_iota(jnp.int32, v.shape, 0)
    return jnp.where(rid < N, v, 0.0)


def _tc_g(parts, x, w):
    def body(p_ref, x_ref, w_ref, o_ref, o16_ref):
        i = pl.program_id(0)
        p = p_ref[:, pl.ds(i * NB, NB)]
        d = jnp.sum(p, axis=0, keepdims=True) + 1.0
        dinv = lax.rsqrt(d)
        h = jnp.dot(
            _rowmask(i, x_ref[...]), w_ref[...], preferred_element_type=jnp.float32
        )
        g = h * _colvec_block(dinv)
        o_ref[...] = g
        o16_ref[...] = g.astype(jnp.bfloat16)

    return pl.pallas_call(
        body,
        grid=(NGRID,),
        in_specs=[
            pl.BlockSpec((NW, NP), lambda i: (0, 0)),
            pl.BlockSpec((NB, 128), lambda i: (i, 0)),
            pl.BlockSpec((128, 128), lambda i: (0, 0)),
        ],
        out_specs=[
            pl.BlockSpec((NB, 128), lambda i: (i, 0)),
            pl.BlockSpec((NB, 128), lambda i: (i, 0)),
        ],
        out_shape=[
            jax.ShapeDtypeStruct((N, 128), jnp.float32),
            jax.ShapeDtypeStruct((N, 128), jnp.bfloat16),
        ],
    )(parts, x, w)


def _tc_out(sparts, g, degparts, bc, pa, wl, bl):
    def body(s_ref, g_ref, p_ref, bc_ref, pa_ref, wl_ref, bl_ref, o_ref):
        i = pl.program_id(0)
        p = p_ref[:, pl.ds(i * NB, NB)]
        d = jnp.sum(p, axis=0, keepdims=True) + 1.0
        dinv = lax.rsqrt(d)
        s = _rowmask(
            i,
            s_ref[0].astype(jnp.float32) + s_ref[1].astype(jnp.float32) + g_ref[...],
        )
        agg = s * _colvec_block(dinv) + bc_ref[...]
        a = pa_ref[0, 0]
        act = jnp.where(agg > 0, agg, a * agg)
        o_ref[...] = (
            jnp.dot(act, wl_ref[...], preferred_element_type=jnp.float32) + bl_ref[...]
        )

    return pl.pallas_call(
        body,
        grid=(NGRID,),
        in_specs=[
            pl.BlockSpec((NC, NB, 128), lambda i: (0, i, 0)),
            pl.BlockSpec((NB, 128), lambda i: (i, 0)),
            pl.BlockSpec((NW, NP), lambda i: (0, 0)),
            pl.BlockSpec((1, 128), lambda i: (0, 0)),
            pl.BlockSpec((1, 1), lambda i: (0, 0)),
            pl.BlockSpec((128, 128), lambda i: (0, 0)),
            pl.BlockSpec((1, 128), lambda i: (0, 0)),
        ],
        out_specs=pl.BlockSpec((NB, 128), lambda i: (i, 0)),
        out_shape=jax.ShapeDtypeStruct((N, 128), jnp.float32),
    )(sparts, g, degparts, bc, pa, wl, bl)


def kernel(x, adj, W_conv, b_conv, prelu_a, W_lin, b_lin):
    row = adj[0]
    col = adj[1]
    degparts = _sc_hist(adj)
    g, g16 = _tc_g(degparts, x, W_conv)
    sparts = _sc_scatter(adj.reshape(2, NW, NITER, CHUNK), g16)
    return _tc_out(
        sparts,
        g,
        degparts,
        b_conv.reshape(1, 128),
        jnp.asarray(prelu_a, jnp.float32).reshape(1, 1),
        W_lin,
        b_lin.reshape(1, 128),
    )
